# baseline (speedup 1.0000x reference)
import jax
import jax.numpy as jnp
from jax.experimental import pallas as pl
from jax.experimental.pallas import tpu as pltpu


def _precompute_body(x_ref, w_ref, b_ref, y_ref, z_ref):
    x = x_ref[...]
    w = w_ref[...]
    f = x.shape[1]
    dn = (((1,), (1,)), ((), ()))
    z_ref[...] = (
        jax.lax.dot_general(x, w[:, :f], dn, preferred_element_type=jnp.float32)
        + b_ref[...]
    )
    y_ref[...] = jax.lax.dot_general(
        x, w[:, f:], dn, preferred_element_type=jnp.float32
    )


def _matmul_body(adj_ref, y_ref, z_ref, out_ref):
    out_ref[...] = z_ref[...] + jnp.dot(
        adj_ref[...], y_ref[...], preferred_element_type=jnp.float32
    )


def _pick_block(n, target):
    for c in range(min(target, n), 7, -1):
        if n % c == 0 and c % 8 == 0:
            return c
    return n


def kernel(x, adj, W, b):
    n, f = x.shape
    y, z = pl.pallas_call(
        _precompute_body,
        out_shape=(
            jax.ShapeDtypeStruct((n, f), jnp.float32),
            jax.ShapeDtypeStruct((n, f), jnp.float32),
        ),
    )(x, W, b.reshape(1, f))

    bm = _pick_block(n, 400)
    out = pl.pallas_call(
        _matmul_body,
        grid=(n // bm,),
        in_specs=[
            pl.BlockSpec((bm, n), lambda i: (i, 0)),
            pl.BlockSpec((n, f), lambda i: (0, 0)),
            pl.BlockSpec((bm, f), lambda i: (i, 0)),
        ],
        out_specs=pl.BlockSpec((bm, f), lambda i: (i, 0)),
        out_shape=jax.ShapeDtypeStruct((n, f), jnp.float32),
        compiler_params=pltpu.CompilerParams(
            dimension_semantics=("arbitrary",)
        ),
    )(adj, y, z)
    return out


# speedup vs baseline: 1.0934x; 1.0934x over previous
import functools

import jax
import jax.numpy as jnp
from jax.experimental import pallas as pl
from jax.experimental.pallas import tpu as pltpu


def _sage_body(bm, adj_ref, x_ref, w_ref, b_ref, out_ref, y_ref):
    i = pl.program_id(0)
    f = x_ref.shape[1]
    dn = (((1,), (1,)), ((), ()))

    @pl.when(i == 0)
    def _():
        y_ref[...] = jax.lax.dot_general(
            x_ref[...], w_ref[:, f:], dn, preferred_element_type=jnp.float32
        )

    xi = x_ref[pl.ds(i * bm, bm), :]
    zi = (
        jax.lax.dot_general(
            xi, w_ref[:, :f], dn, preferred_element_type=jnp.float32
        )
        + b_ref[...]
    )
    out_ref[...] = zi + jnp.dot(
        adj_ref[...], y_ref[...], preferred_element_type=jnp.float32
    )


def _pick_block(n, target):
    for c in range(min(target, n), 7, -1):
        if n % c == 0 and c % 8 == 0:
            return c
    return n


def kernel(x, adj, W, b):
    n, f = x.shape
    bm = _pick_block(n, 400)
    out = pl.pallas_call(
        functools.partial(_sage_body, bm),
        grid=(n // bm,),
        in_specs=[
            pl.BlockSpec((bm, n), lambda i: (i, 0)),
            pl.BlockSpec((n, f), lambda i: (0, 0)),
            pl.BlockSpec(W.shape, lambda i: (0, 0)),
            pl.BlockSpec((1, f), lambda i: (0, 0)),
        ],
        out_specs=pl.BlockSpec((bm, f), lambda i: (i, 0)),
        out_shape=jax.ShapeDtypeStruct((n, f), jnp.float32),
        scratch_shapes=[pltpu.VMEM((n, f), jnp.float32)],
        compiler_params=pltpu.CompilerParams(
            dimension_semantics=("arbitrary",)
        ),
    )(adj, x, W, b.reshape(1, f))
    return out


# speedup vs baseline: 1.1019x; 1.0078x over previous
import functools

import jax
import jax.numpy as jnp
from jax.experimental import pallas as pl
from jax.experimental.pallas import tpu as pltpu


def _sage_body(bm, adj_ref, x_ref, w_ref, b_ref, out_ref, y_ref):
    i = pl.program_id(0)
    f = x_ref.shape[1]
    dn = (((1,), (1,)), ((), ()))

    @pl.when(i == 0)
    def _():
        y_ref[...] = jax.lax.dot_general(
            x_ref[...], w_ref[:, f:], dn, preferred_element_type=jnp.float32
        )

    xi = x_ref[pl.ds(i * bm, bm), :]
    zi = (
        jax.lax.dot_general(
            xi, w_ref[:, :f], dn, preferred_element_type=jnp.float32
        )
        + b_ref[...]
    )
    out_ref[...] = zi + jnp.dot(
        adj_ref[...], y_ref[...], preferred_element_type=jnp.float32
    )


def _pick_block(n, target):
    for c in range(min(target, n), 7, -1):
        if n % c == 0 and c % 8 == 0:
            return c
    return n


def kernel(x, adj, W, b):
    n, f = x.shape
    bm = _pick_block(n, 200)
    out = pl.pallas_call(
        functools.partial(_sage_body, bm),
        grid=(n // bm,),
        in_specs=[
            pl.BlockSpec((bm, n), lambda i: (i, 0)),
            pl.BlockSpec((n, f), lambda i: (0, 0)),
            pl.BlockSpec(W.shape, lambda i: (0, 0)),
            pl.BlockSpec((1, f), lambda i: (0, 0)),
        ],
        out_specs=pl.BlockSpec((bm, f), lambda i: (i, 0)),
        out_shape=jax.ShapeDtypeStruct((n, f), jnp.float32),
        scratch_shapes=[pltpu.VMEM((n, f), jnp.float32)],
        compiler_params=pltpu.CompilerParams(
            dimension_semantics=("arbitrary",)
        ),
    )(adj, x, W, b.reshape(1, f))
    return out
